# Initial kernel scaffold; baseline (speedup 1.0000x reference)
import functools

import jax
import jax.numpy as jnp
from jax import lax
from jax.experimental import pallas as pl
from jax.experimental.pallas import tpu as pltpu
from jax.experimental.pallas import tpu_sc as plsc

LANES = 16


def _eproj_call(edge_attr, Weh, E, DE, HALF):
    BE = 2000

    def body(ea_ref, we_ref, out_ref):
        out_ref[0] = jnp.dot(ea_ref[...], we_ref[0],
                             preferred_element_type=jnp.float32)

    return pl.pallas_call(
        body,
        grid=(2, E // BE),
        in_specs=[
            pl.BlockSpec((BE, DE), lambda h, i: (i, 0)),
            pl.BlockSpec((1, DE, HALF), lambda h, i: (h, 0, 0)),
        ],
        out_specs=pl.BlockSpec((1, BE, HALF), lambda h, i: (h, i, 0)),
        out_shape=jax.ShapeDtypeStruct((2, E, HALF), jnp.float32),
    )(edge_attr, Weh)


def _sc_agg_call(x2, ep2, src, dst2d, N, E, HALF):
    NS = 16
    EperS = E // NS
    K = 80
    NCHUNK = EperS // K
    NperS = N // NS

    mesh = plsc.VectorSubcoreMesh(core_axis_name="c", subcore_axis_name="s")

    @functools.partial(
        pl.kernel,
        out_type=jax.ShapeDtypeStruct((2 * N, HALF), jnp.float32),
        mesh=mesh,
        scratch_types=[
            pltpu.VMEM((EperS,), jnp.int32),
            pltpu.VMEM((NCHUNK, K), jnp.int32),
            pltpu.VMEM((K, HALF), jnp.float32),
            pltpu.VMEM((K, HALF), jnp.float32),
            pltpu.VMEM_SHARED((N, HALF), jnp.float32),
            pltpu.SemaphoreType.DMA,
        ],
    )
    def sc_agg(x2_hbm, ep_hbm, src_hbm, dst_hbm, out_hbm,
               srcv, dstv, xbuf, ebuf, agg_sh, sem):
        c = lax.axis_index("c")
        s = lax.axis_index("s")
        ebase = s * EperS

        pltpu.sync_copy(src_hbm.at[pl.ds(ebase, EperS)], srcv)

        def shift_body(j, carry):
            sl = pl.ds(j * LANES, LANES)
            srcv[sl] = srcv[sl] * 2 + c
            return carry

        lax.fori_loop(0, EperS // LANES, shift_body, 0, unroll=8)

        pltpu.sync_copy(dst_hbm.at[pl.ds(s * NCHUNK, NCHUNK)], dstv)

        def zrow(r, carry):
            for j in range(HALF // LANES):
                xbuf[r, pl.ds(j * LANES, LANES)] = jnp.zeros((LANES,),
                                                             jnp.float32)
            return carry

        lax.fori_loop(0, K, zrow, 0)
        nfull = NperS // K
        rem = NperS - nfull * K
        for t in range(nfull):
            pltpu.sync_copy(xbuf, agg_sh.at[pl.ds(s * NperS + t * K, K)])
        if rem:
            pltpu.sync_copy(xbuf.at[pl.ds(0, rem)],
                            agg_sh.at[pl.ds(s * NperS + nfull * K, rem)])
        plsc.subcore_barrier()

        def chunk(i, carry):
            pltpu.async_copy(x2_hbm.at[srcv.at[pl.ds(i * K, K)]], xbuf,
                             sem).wait()
            pltpu.sync_copy(ep_hbm.at[pl.ds(c * E + ebase + i * K, K)], ebuf)

            def row(r, rcarry):
                for j in range(HALF // LANES):
                    sl = pl.ds(j * LANES, LANES)
                    v = xbuf[r, sl] + ebuf[r, sl]
                    xbuf[r, sl] = jnp.maximum(v, 0.0)
                return rcarry

            lax.fori_loop(0, K, row, 0)
            pltpu.sync_copy(xbuf, agg_sh.at[dstv.at[i]], add=True)
            return carry

        lax.fori_loop(0, NCHUNK, chunk, 0)
        plsc.subcore_barrier()

        pltpu.sync_copy(agg_sh.at[pl.ds(s * NperS, NperS)],
                        out_hbm.at[pl.ds(c * N + s * NperS, NperS)])

    return sc_agg(x2, ep2, src, dst2d)


def _mlp_call(x, agg, Wc, bc2, W1, b12, W2, b22, N, DF, HALF, DC, DH):
    BN = 400

    def body(x_ref, g0_ref, g1_ref, wc_ref, bc_ref, w1_ref, b1_ref,
             w2_ref, b2_ref, out_ref):
        a0 = x_ref[:, :HALF] + g0_ref[...]
        a1 = x_ref[:, HALF:] + g1_ref[...]
        h = (jnp.dot(a0, wc_ref[:HALF, :], preferred_element_type=jnp.float32)
             + jnp.dot(a1, wc_ref[HALF:, :],
                       preferred_element_type=jnp.float32)
             + bc_ref[...])
        h = jnp.maximum(
            jnp.dot(h, w1_ref[...], preferred_element_type=jnp.float32)
            + b1_ref[...], 0.0)
        out_ref[...] = (jnp.dot(h, w2_ref[...],
                                preferred_element_type=jnp.float32)
                        + b2_ref[...])

    nb = N // BN
    return pl.pallas_call(
        body,
        grid=(nb,),
        in_specs=[
            pl.BlockSpec((BN, DF), lambda i: (i, 0)),
            pl.BlockSpec((BN, HALF), lambda i: (i, 0)),
            pl.BlockSpec((BN, HALF), lambda i, _nb=nb: (i + _nb, 0)),
            pl.BlockSpec((DF, DC), lambda i: (0, 0)),
            pl.BlockSpec((1, DC), lambda i: (0, 0)),
            pl.BlockSpec((DC, DH), lambda i: (0, 0)),
            pl.BlockSpec((1, DH), lambda i: (0, 0)),
            pl.BlockSpec((DH, DF), lambda i: (0, 0)),
            pl.BlockSpec((1, DF), lambda i: (0, 0)),
        ],
        out_specs=pl.BlockSpec((BN, DF), lambda i: (i, 0)),
        out_shape=jax.ShapeDtypeStruct((N, DF), jnp.float32),
    )(x, agg, agg, Wc, bc2, W1, b12, W2, b22)


def kernel(x, edge_index, edge_attr, We, Wc, bc, W1, b1, W2, b2):
    N, DF = x.shape
    E = edge_index.shape[1]
    DE = edge_attr.shape[1]
    DC = Wc.shape[1]
    DH = W1.shape[1]
    HALF = DF // 2

    src = edge_index[0]
    dst = edge_index[1]

    x2 = x.reshape(2 * N, HALF)
    Weh = We.reshape(DE, 2, HALF).transpose(1, 0, 2)
    dst2d = dst.reshape(E // 80, 80)

    ep = _eproj_call(edge_attr, Weh, E, DE, HALF)
    ep2 = ep.reshape(2 * E, HALF)

    agg = _sc_agg_call(x2, ep2, src, dst2d, N, E, HALF)

    out = _mlp_call(x, agg, Wc, bc.reshape(1, DC), W1, b1.reshape(1, DH),
                    W2, b2.reshape(1, DF), N, DF, HALF, DC, DH)
    return out


# speedup vs baseline: 1.9169x; 1.9169x over previous
import functools

import jax
import jax.numpy as jnp
from jax import lax
from jax.experimental import pallas as pl
from jax.experimental.pallas import tpu as pltpu
from jax.experimental.pallas import tpu_sc as plsc

LANES = 16


def _eproj_call(edge_attr, Weh, E, DE, HALF):
    BE = 2000

    def body(ea_ref, we_ref, out_ref):
        out_ref[0] = jnp.dot(ea_ref[...], we_ref[0],
                             preferred_element_type=jnp.float32)

    return pl.pallas_call(
        body,
        grid=(2, E // BE),
        in_specs=[
            pl.BlockSpec((BE, DE), lambda h, i: (i, 0)),
            pl.BlockSpec((1, DE, HALF), lambda h, i: (h, 0, 0)),
        ],
        out_specs=pl.BlockSpec((1, BE, HALF), lambda h, i: (h, i, 0)),
        out_shape=jax.ShapeDtypeStruct((2, E, HALF), jnp.float32),
    )(edge_attr, Weh)


def _sc_agg_call(x2, ep2, src, dst2d, N, E, HALF):
    NS = 16
    EperS = E // NS
    K = 80
    NCHUNK = EperS // K
    NperS = N // NS

    mesh = plsc.VectorSubcoreMesh(core_axis_name="c", subcore_axis_name="s")

    @functools.partial(
        pl.kernel,
        out_type=jax.ShapeDtypeStruct((2 * N, HALF), jnp.float32),
        mesh=mesh,
        scratch_types=[
            pltpu.VMEM((EperS,), jnp.int32),
            pltpu.VMEM((NCHUNK, K), jnp.int32),
            pltpu.VMEM((K, HALF), jnp.float32),
            pltpu.VMEM((K, HALF), jnp.float32),
            pltpu.VMEM_SHARED((N, HALF), jnp.float32),
            pltpu.SemaphoreType.DMA,
        ],
    )
    def sc_agg(x2_hbm, ep_hbm, src_hbm, dst_hbm, out_hbm,
               srcv, dstv, xbuf, ebuf, agg_sh, sem):
        c = lax.axis_index("c")
        s = lax.axis_index("s")
        ebase = s * EperS

        pltpu.sync_copy(src_hbm.at[pl.ds(ebase, EperS)], srcv)

        def shift_body(j, carry):
            sl = pl.ds(j * LANES, LANES)
            srcv[sl] = srcv[sl] * 2 + c
            return carry

        lax.fori_loop(0, EperS // LANES, shift_body, 0, unroll=8)

        pltpu.sync_copy(dst_hbm.at[s], dstv)

        def zrow(r, carry):
            for j in range(HALF // LANES):
                xbuf[r, pl.ds(j * LANES, LANES)] = jnp.zeros((LANES,),
                                                             jnp.float32)
            return carry

        lax.fori_loop(0, K, zrow, 0)
        nchunk_n = N // K
        for t in range(-(-nchunk_n // NS)):
            idx = s + NS * t

            @pl.when(idx < nchunk_n)
            def _():
                pltpu.sync_copy(xbuf, agg_sh.at[pl.ds(idx * K, K)])

        plsc.subcore_barrier()

        def chunk(i, carry):
            pltpu.async_copy(x2_hbm.at[srcv.at[pl.ds(i * K, K)]], xbuf,
                             sem).wait()
            pltpu.sync_copy(ep_hbm.at[pl.ds(c * E + ebase + i * K, K)], ebuf)

            def row(r, rcarry):
                for j in range(HALF // LANES):
                    sl = pl.ds(j * LANES, LANES)
                    v = xbuf[r, sl] + ebuf[r, sl]
                    xbuf[r, sl] = jnp.maximum(v, 0.0)
                return rcarry

            lax.fori_loop(0, K, row, 0)
            pltpu.sync_copy(xbuf, agg_sh.at[dstv.at[i]], add=True)
            return carry

        lax.fori_loop(0, NCHUNK, chunk, 0)
        plsc.subcore_barrier()

        for t in range(-(-nchunk_n // NS)):
            idx = s + NS * t

            @pl.when(idx < nchunk_n)
            def _():
                pltpu.sync_copy(agg_sh.at[pl.ds(idx * K, K)],
                                out_hbm.at[pl.ds(c * N + idx * K, K)])

    return sc_agg(x2, ep2, src, dst2d)


def _mlp_call(x, agg, Wc, bc2, W1, b12, W2, b22, N, DF, HALF, DC, DH):
    BN = 400

    def body(x_ref, g0_ref, g1_ref, wc_ref, bc_ref, w1_ref, b1_ref,
             w2_ref, b2_ref, out_ref):
        a0 = x_ref[:, :HALF] + g0_ref[...]
        a1 = x_ref[:, HALF:] + g1_ref[...]
        h = (jnp.dot(a0, wc_ref[:HALF, :], preferred_element_type=jnp.float32)
             + jnp.dot(a1, wc_ref[HALF:, :],
                       preferred_element_type=jnp.float32)
             + bc_ref[...])
        h = jnp.maximum(
            jnp.dot(h, w1_ref[...], preferred_element_type=jnp.float32)
            + b1_ref[...], 0.0)
        out_ref[...] = (jnp.dot(h, w2_ref[...],
                                preferred_element_type=jnp.float32)
                        + b2_ref[...])

    nb = N // BN
    return pl.pallas_call(
        body,
        grid=(nb,),
        in_specs=[
            pl.BlockSpec((BN, DF), lambda i: (i, 0)),
            pl.BlockSpec((BN, HALF), lambda i: (i, 0)),
            pl.BlockSpec((BN, HALF), lambda i, _nb=nb: (i + _nb, 0)),
            pl.BlockSpec((DF, DC), lambda i: (0, 0)),
            pl.BlockSpec((1, DC), lambda i: (0, 0)),
            pl.BlockSpec((DC, DH), lambda i: (0, 0)),
            pl.BlockSpec((1, DH), lambda i: (0, 0)),
            pl.BlockSpec((DH, DF), lambda i: (0, 0)),
            pl.BlockSpec((1, DF), lambda i: (0, 0)),
        ],
        out_specs=pl.BlockSpec((BN, DF), lambda i: (i, 0)),
        out_shape=jax.ShapeDtypeStruct((N, DF), jnp.float32),
    )(x, agg, agg, Wc, bc2, W1, b12, W2, b22)


def kernel(x, edge_index, edge_attr, We, Wc, bc, W1, b1, W2, b2):
    N, DF = x.shape
    E = edge_index.shape[1]
    DE = edge_attr.shape[1]
    DC = Wc.shape[1]
    DH = W1.shape[1]
    HALF = DF // 2

    src = edge_index[0]
    dst = edge_index[1]

    x2 = x.reshape(2 * N, HALF)
    Weh = We.reshape(DE, 2, HALF).transpose(1, 0, 2)
    dst3d = dst.reshape(16, E // (16 * 80), 80)

    ep = _eproj_call(edge_attr, Weh, E, DE, HALF)
    ep2 = ep.reshape(2 * E, HALF)

    agg = _sc_agg_call(x2, ep2, src, dst3d, N, E, HALF)

    out = _mlp_call(x, agg, Wc, bc.reshape(1, DC), W1, b1.reshape(1, DH),
                    W2, b2.reshape(1, DF), N, DF, HALF, DC, DH)
    return out
